# baseline (speedup 1.0000x reference)
import jax
import jax.numpy as jnp
from jax.experimental import pallas as pl

_BLK = 1024


def _fused_body(x_ref, wbd_ref, bvec_ref, wl2_ref, bl_ref, o_ref):
    a = (
        jnp.dot(x_ref[:], wbd_ref[:], preferred_element_type=jnp.float32)
        + bvec_ref[:]
    )
    u = jax.nn.sigmoid(a)
    v = u + u - 1.0
    h = jnp.maximum(u * jnp.roll(v, -32, axis=1), 0.0)
    o_ref[:] = (
        jnp.dot(h, wl2_ref[:], preferred_element_type=jnp.float32) + bl_ref[:]
    )


def kernel(x, edge_index, edge_weight, W_xz, b_xz, W_hz, b_hz, W_xr, b_xr,
           W_hr, b_hr, W_xh, b_xh, W_hh, b_hh, W_lin, b_lin):
    n, f = x.shape
    hid = W_xz.shape[1]
    half = n // 2
    xr = x.reshape(half, 2 * f)

    zpad = jnp.zeros((f, 2 * hid), jnp.float32)
    top = jnp.concatenate([-W_xz, 2.0 * W_xh, zpad], axis=1)
    bot = jnp.concatenate([zpad, -W_xz, 2.0 * W_xh], axis=1)
    wbd = jnp.concatenate([top, bot], axis=0)

    bz = -(b_xz + b_hz)
    bh = 2.0 * (b_xh + b_hh)
    bvec = jnp.concatenate([bz, bh, bz, bh]).reshape(1, 4 * hid)

    zcol = jnp.zeros((hid, 1), jnp.float32)
    wl2 = jnp.concatenate(
        [
            jnp.concatenate([W_lin, zcol], axis=1),
            jnp.zeros((hid, 2), jnp.float32),
            jnp.concatenate([zcol, W_lin], axis=1),
            jnp.zeros((hid, 2), jnp.float32),
        ],
        axis=0,
    )
    bl = jnp.broadcast_to(b_lin.reshape(1, 1), (1, 2))

    out2 = pl.pallas_call(
        _fused_body,
        grid=(pl.cdiv(half, _BLK),),
        in_specs=[
            pl.BlockSpec((_BLK, 2 * f), lambda i: (i, 0)),
            pl.BlockSpec((2 * f, 4 * hid), lambda i: (0, 0)),
            pl.BlockSpec((1, 4 * hid), lambda i: (0, 0)),
            pl.BlockSpec((4 * hid, 2), lambda i: (0, 0)),
            pl.BlockSpec((1, 2), lambda i: (0, 0)),
        ],
        out_specs=pl.BlockSpec((_BLK, 2), lambda i: (i, 0)),
        out_shape=jax.ShapeDtypeStruct((half, 2), jnp.float32),
    )(xr, wbd, bvec, wl2, bl)
    return out2.reshape(n, 1)


# speedup vs baseline: 1.0560x; 1.0560x over previous
import jax
import jax.numpy as jnp
from jax.experimental import pallas as pl

_BLK = 1024


def _fused_body(x_ref, wz_ref, wh_ref, bxz_ref, bhz_ref, bxh_ref, bhh_ref,
                wl_ref, bl_ref, o_ref):
    f = wz_ref.shape[0]
    hid = wz_ref.shape[1]
    w = jnp.concatenate([-wz_ref[:], 2.0 * wh_ref[:]], axis=1)
    bv = jnp.concatenate(
        [-(bxz_ref[:] + bhz_ref[:]), 2.0 * (bxh_ref[:] + bhh_ref[:])], axis=1
    )
    xe = x_ref[:, :f]
    xo = x_ref[:, f:]
    ae = jnp.dot(xe, w, preferred_element_type=jnp.float32) + bv
    ao = jnp.dot(xo, w, preferred_element_type=jnp.float32) + bv
    a = jnp.concatenate([ae, ao], axis=1)
    u = jax.nn.sigmoid(a)
    v = u + u - 1.0
    h = jnp.maximum(u * jnp.roll(v, -hid, axis=1), 0.0)
    wl = wl_ref[:]
    zc = jnp.zeros((hid, 1), jnp.float32)
    wl2 = jnp.concatenate(
        [
            jnp.concatenate([wl, zc, zc, zc], axis=0),
            jnp.concatenate([zc, zc, wl, zc], axis=0),
        ],
        axis=1,
    )
    o_ref[:] = (
        jnp.dot(h, wl2, preferred_element_type=jnp.float32) + bl_ref[0, 0]
    )


def kernel(x, edge_index, edge_weight, W_xz, b_xz, W_hz, b_hz, W_xr, b_xr,
           W_hr, b_hr, W_xh, b_xh, W_hh, b_hh, W_lin, b_lin):
    n, f = x.shape
    hid = W_xz.shape[1]
    half = n // 2
    xr = x.reshape(half, 2 * f)
    _vec = pl.BlockSpec((1, hid), lambda i: (0, 0))
    out2 = pl.pallas_call(
        _fused_body,
        grid=(pl.cdiv(half, _BLK),),
        in_specs=[
            pl.BlockSpec((_BLK, 2 * f), lambda i: (i, 0)),
            pl.BlockSpec((f, hid), lambda i: (0, 0)),
            pl.BlockSpec((f, hid), lambda i: (0, 0)),
            _vec, _vec, _vec, _vec,
            pl.BlockSpec((hid, 1), lambda i: (0, 0)),
            pl.BlockSpec((1, 1), lambda i: (0, 0)),
        ],
        out_specs=pl.BlockSpec((_BLK, 2), lambda i: (i, 0)),
        out_shape=jax.ShapeDtypeStruct((half, 2), jnp.float32),
    )(xr, W_xz, W_xh, b_xz.reshape(1, hid), b_hz.reshape(1, hid),
      b_xh.reshape(1, hid), b_hh.reshape(1, hid), W_lin, b_lin.reshape(1, 1))
    return out2.reshape(n, 1)


# speedup vs baseline: 1.4119x; 1.3370x over previous
import jax
import jax.numpy as jnp
from jax.experimental import pallas as pl

_BLK = 2048


def _fused_body(x_ref, wz_ref, wh_ref, bxz_ref, bhz_ref, bxh_ref, bhh_ref,
                wl_ref, bl_ref, o_ref):
    hid = wz_ref.shape[1]
    w = jnp.concatenate([-wz_ref[:], 2.0 * wh_ref[:]], axis=1)
    bv = jnp.concatenate(
        [-(bxz_ref[:] + bhz_ref[:]), 2.0 * (bxh_ref[:] + bhh_ref[:])], axis=1
    )
    a = jnp.dot(x_ref[:], w, preferred_element_type=jnp.float32) + bv
    u = jax.nn.sigmoid(a)
    v = u + u - 1.0
    h = jnp.maximum(u * jnp.roll(v, -hid, axis=1), 0.0)
    wl2 = jnp.concatenate(
        [wl_ref[:], jnp.zeros((hid, 1), jnp.float32)], axis=0
    )
    o_ref[:] = (
        jnp.dot(h, wl2, preferred_element_type=jnp.float32) + bl_ref[0, 0]
    )


def kernel(x, edge_index, edge_weight, W_xz, b_xz, W_hz, b_hz, W_xr, b_xr,
           W_hr, b_hr, W_xh, b_xh, W_hh, b_hh, W_lin, b_lin):
    n, f = x.shape
    hid = W_xz.shape[1]
    _vec = pl.BlockSpec((1, hid), lambda i: (0, 0))
    out = pl.pallas_call(
        _fused_body,
        grid=(pl.cdiv(n, _BLK),),
        in_specs=[
            pl.BlockSpec((_BLK, f), lambda i: (i, 0)),
            pl.BlockSpec((f, hid), lambda i: (0, 0)),
            pl.BlockSpec((f, hid), lambda i: (0, 0)),
            _vec, _vec, _vec, _vec,
            pl.BlockSpec((hid, 1), lambda i: (0, 0)),
            pl.BlockSpec((1, 1), lambda i: (0, 0)),
        ],
        out_specs=pl.BlockSpec((_BLK, 1), lambda i: (i, 0)),
        out_shape=jax.ShapeDtypeStruct((n, 1), jnp.float32),
    )(x, W_xz, W_xh, b_xz.reshape(1, hid), b_hz.reshape(1, hid),
      b_xh.reshape(1, hid), b_hh.reshape(1, hid), W_lin, b_lin.reshape(1, 1))
    return out


# speedup vs baseline: 1.7376x; 1.2306x over previous
import jax
import jax.numpy as jnp
from jax.experimental import pallas as pl

_BLK = 2048


def _fused_body(x_ref, wz_ref, wh_ref, bxz_ref, bhz_ref, bxh_ref, bhh_ref,
                wl_ref, bl_ref, o_ref):
    hid = wz_ref.shape[1]
    w = jnp.concatenate([-wz_ref[:], 2.0 * wh_ref[:]], axis=1)
    bv = jnp.concatenate(
        [-(bxz_ref[:] + bhz_ref[:]), 2.0 * (bxh_ref[:] + bhh_ref[:])], axis=1
    )
    a = jnp.dot(x_ref[:], w, preferred_element_type=jnp.float32) + bv
    u = jax.nn.sigmoid(a)
    v = u + u - 1.0
    h = jnp.maximum(u * jnp.roll(v, -hid, axis=1), 0.0)
    wl2 = jnp.concatenate(
        [wl_ref[:], jnp.zeros((hid, 1), jnp.float32)], axis=0
    )
    col = jnp.dot(h, wl2, preferred_element_type=jnp.float32) + bl_ref[0, 0]
    o_ref[:] = col.reshape(o_ref.shape)


def kernel(x, edge_index, edge_weight, W_xz, b_xz, W_hz, b_hz, W_xr, b_xr,
           W_hr, b_hr, W_xh, b_xh, W_hh, b_hh, W_lin, b_lin):
    n, f = x.shape
    hid = W_xz.shape[1]
    rows = _BLK // 128
    nsteps = pl.cdiv(n, _BLK)
    _vec = pl.BlockSpec((1, hid), lambda i: (0, 0))
    out_t = pl.pallas_call(
        _fused_body,
        grid=(nsteps,),
        in_specs=[
            pl.BlockSpec((_BLK, f), lambda i: (i, 0)),
            pl.BlockSpec((f, hid), lambda i: (0, 0)),
            pl.BlockSpec((f, hid), lambda i: (0, 0)),
            _vec, _vec, _vec, _vec,
            pl.BlockSpec((hid, 1), lambda i: (0, 0)),
            pl.BlockSpec((1, 1), lambda i: (0, 0)),
        ],
        out_specs=pl.BlockSpec((rows, 128), lambda i: (i, 0)),
        out_shape=jax.ShapeDtypeStruct((nsteps * rows, 128), jnp.float32),
    )(x, W_xz, W_xh, b_xz.reshape(1, hid), b_hz.reshape(1, hid),
      b_xh.reshape(1, hid), b_hh.reshape(1, hid), W_lin, b_lin.reshape(1, 1))
    return out_t.reshape(nsteps * _BLK, 1)[:n]
